# baseline (speedup 1.0000x reference)
import jax
import jax.numpy as jnp
from jax.experimental import pallas as pl
from jax.experimental.pallas import tpu as pltpu

_CHUNKS = 8
_NBUF = 2
_LOOKAHEAD = 1


def _body(x_hbm, o_hbm, off_hbm, buf, off_smem, in_sems, out_sems, off_sem):
    n_rows = x_hbm.shape[0]
    b = off_hbm.shape[0] - 1
    seq_len = n_rows // b
    for i in range(b + 1):
        off_smem[i] = i * seq_len
    off_copy = pltpu.make_async_copy(off_smem, off_hbm, off_sem)
    off_copy.start()

    rows = n_rows // _CHUNKS
    ins = [
        pltpu.make_async_copy(
            x_hbm.at[pl.ds(k * rows, rows)], buf.at[k % _NBUF],
            in_sems.at[k % _NBUF],
        )
        for k in range(_CHUNKS)
    ]
    outs = [
        pltpu.make_async_copy(
            buf.at[k % _NBUF], o_hbm.at[pl.ds(k * rows, rows)],
            out_sems.at[k % _NBUF],
        )
        for k in range(_CHUNKS)
    ]
    for k in range(_LOOKAHEAD):
        ins[k].start()
    for k in range(_CHUNKS):
        if k >= _LOOKAHEAD:
            outs[k - _LOOKAHEAD].wait()
        if k + _LOOKAHEAD < _CHUNKS:
            ins[k + _LOOKAHEAD].start()
        ins[k].wait()
        outs[k].start()
    for k in range(_CHUNKS - _LOOKAHEAD, _CHUNKS):
        outs[k].wait()
    off_copy.wait()


def kernel(X):
    B, L, d = X.shape
    x_flat = X.reshape(B * L, d)
    n_rows = B * L
    rows = n_rows // _CHUNKS
    values, offsets = pl.pallas_call(
        _body,
        in_specs=[pl.BlockSpec(memory_space=pl.ANY)],
        out_specs=[
            pl.BlockSpec(memory_space=pl.ANY),
            pl.BlockSpec(memory_space=pl.ANY),
        ],
        out_shape=[
            jax.ShapeDtypeStruct((n_rows, d), x_flat.dtype),
            jax.ShapeDtypeStruct((B + 1,), jnp.int32),
        ],
        scratch_shapes=[
            pltpu.VMEM((_NBUF, rows, d), x_flat.dtype),
            pltpu.SMEM((B + 1,), jnp.int32),
            pltpu.SemaphoreType.DMA((_NBUF,)),
            pltpu.SemaphoreType.DMA((_NBUF,)),
            pltpu.SemaphoreType.DMA,
        ],
    )(x_flat)
    return (values, offsets)


# speedup vs baseline: 1.0062x; 1.0062x over previous
import jax
import jax.numpy as jnp
from jax.experimental import pallas as pl
from jax.experimental.pallas import tpu as pltpu

_CHUNKS = 32
_NBUF = 8
_LOOKAHEAD = 4


def _body(x_hbm, o_hbm, off_hbm, buf, off_smem, in_sems, out_sems, off_sem):
    n_rows = x_hbm.shape[0]
    b = off_hbm.shape[0] - 1
    seq_len = n_rows // b
    for i in range(b + 1):
        off_smem[i] = i * seq_len
    off_copy = pltpu.make_async_copy(off_smem, off_hbm, off_sem)
    off_copy.start()

    rows = n_rows // _CHUNKS
    ins = [
        pltpu.make_async_copy(
            x_hbm.at[pl.ds(k * rows, rows)], buf.at[k % _NBUF],
            in_sems.at[k % _NBUF],
        )
        for k in range(_CHUNKS)
    ]
    outs = [
        pltpu.make_async_copy(
            buf.at[k % _NBUF], o_hbm.at[pl.ds(k * rows, rows)],
            out_sems.at[k % _NBUF],
        )
        for k in range(_CHUNKS)
    ]
    for k in range(_LOOKAHEAD):
        ins[k].start()
    for k in range(_CHUNKS):
        if k >= _LOOKAHEAD:
            outs[k - _LOOKAHEAD].wait()
        if k + _LOOKAHEAD < _CHUNKS:
            ins[k + _LOOKAHEAD].start()
        ins[k].wait()
        outs[k].start()
    for k in range(_CHUNKS - _LOOKAHEAD, _CHUNKS):
        outs[k].wait()
    off_copy.wait()


def kernel(X):
    B, L, d = X.shape
    x_flat = X.reshape(B * L, d)
    n_rows = B * L
    rows = n_rows // _CHUNKS
    values, offsets = pl.pallas_call(
        _body,
        in_specs=[pl.BlockSpec(memory_space=pl.ANY)],
        out_specs=[
            pl.BlockSpec(memory_space=pl.ANY),
            pl.BlockSpec(memory_space=pl.ANY),
        ],
        out_shape=[
            jax.ShapeDtypeStruct((n_rows, d), x_flat.dtype),
            jax.ShapeDtypeStruct((B + 1,), jnp.int32),
        ],
        scratch_shapes=[
            pltpu.VMEM((_NBUF, rows, d), x_flat.dtype),
            pltpu.SMEM((B + 1,), jnp.int32),
            pltpu.SemaphoreType.DMA((_NBUF,)),
            pltpu.SemaphoreType.DMA((_NBUF,)),
            pltpu.SemaphoreType.DMA,
        ],
    )(x_flat)
    return (values, offsets)


# speedup vs baseline: 1.0118x; 1.0056x over previous
import jax
import jax.numpy as jnp
from jax.experimental import pallas as pl
from jax.experimental.pallas import tpu as pltpu

_CHUNKS = 16
_NBUF = 6
_LOOKAHEAD = 3


def _body(x_hbm, o_hbm, off_hbm, buf, off_smem, in_sems, out_sems, off_sem):
    n_rows = x_hbm.shape[0]
    b = off_hbm.shape[0] - 1
    seq_len = n_rows // b
    for i in range(b + 1):
        off_smem[i] = i * seq_len
    off_copy = pltpu.make_async_copy(off_smem, off_hbm, off_sem)
    off_copy.start()

    rows = n_rows // _CHUNKS
    ins = [
        pltpu.make_async_copy(
            x_hbm.at[pl.ds(k * rows, rows)], buf.at[k % _NBUF],
            in_sems.at[k % _NBUF],
        )
        for k in range(_CHUNKS)
    ]
    outs = [
        pltpu.make_async_copy(
            buf.at[k % _NBUF], o_hbm.at[pl.ds(k * rows, rows)],
            out_sems.at[k % _NBUF],
        )
        for k in range(_CHUNKS)
    ]
    for k in range(_LOOKAHEAD):
        ins[k].start()
    for k in range(_CHUNKS):
        if k >= _LOOKAHEAD:
            outs[k - _LOOKAHEAD].wait()
        if k + _LOOKAHEAD < _CHUNKS:
            ins[k + _LOOKAHEAD].start()
        ins[k].wait()
        outs[k].start()
    for k in range(_CHUNKS - _LOOKAHEAD, _CHUNKS):
        outs[k].wait()
    off_copy.wait()


def kernel(X):
    B, L, d = X.shape
    x_flat = X.reshape(B * L, d)
    n_rows = B * L
    rows = n_rows // _CHUNKS
    values, offsets = pl.pallas_call(
        _body,
        in_specs=[pl.BlockSpec(memory_space=pl.ANY)],
        out_specs=[
            pl.BlockSpec(memory_space=pl.ANY),
            pl.BlockSpec(memory_space=pl.ANY),
        ],
        out_shape=[
            jax.ShapeDtypeStruct((n_rows, d), x_flat.dtype),
            jax.ShapeDtypeStruct((B + 1,), jnp.int32),
        ],
        scratch_shapes=[
            pltpu.VMEM((_NBUF, rows, d), x_flat.dtype),
            pltpu.SMEM((B + 1,), jnp.int32),
            pltpu.SemaphoreType.DMA((_NBUF,)),
            pltpu.SemaphoreType.DMA((_NBUF,)),
            pltpu.SemaphoreType.DMA,
        ],
    )(x_flat)
    return (values, offsets)
